# Initial kernel scaffold; baseline (speedup 1.0000x reference)
import jax
import jax.numpy as jnp
from jax.experimental import pallas as pl


def kernel(x_nchw):
    raise NotImplementedError("write your pallas kernel here")


# speedup vs baseline: 1.0291x; 1.0291x over previous
import functools

import jax
import jax.numpy as jnp
from jax.experimental import pallas as pl
from jax.experimental.pallas import tpu as pltpu


def _gct_block(x_ref, o_ref, *, c_gate, eps_s):
    x = x_ref[...].astype(jnp.float32)
    inv_c = 1.0 / x.shape[1]
    s = jnp.sum(x, axis=2, keepdims=True)
    m1 = jnp.sum(s, axis=1, keepdims=True) * inv_c
    m2 = jnp.sum(s * s, axis=1, keepdims=True) * inv_c
    var = jnp.maximum(m2 - m1 * m1, 0.0)
    z = (s - m1) * jax.lax.rsqrt(var + eps_s)
    o_ref[...] = (x * jnp.exp(z * z * (-0.5 * c_gate))).astype(o_ref.dtype)


@functools.partial(jax.jit, static_argnames=("c_gate", "eps"))
def _gct_forward(x_nchw, *, c_gate=2.0, eps=1e-5):
    n, c, h, w = x_nchw.shape
    hw = h * w
    x3 = x_nchw.reshape(n, c, hw)
    slab = c * hw * jnp.dtype(x_nchw.dtype).itemsize

    budget = 40 << 20
    nb = 1
    for cand in range(n, 0, -1):
        if n % cand == 0 and 4 * cand * slab <= budget:
            nb = cand
            break

    vmem_limit = int(min(60 << 20, 4 * nb * slab + (8 << 20)))
    y3 = pl.pallas_call(
        functools.partial(_gct_block, c_gate=float(c_gate),
                          eps_s=float(eps) * float(hw) * float(hw)),
        out_shape=jax.ShapeDtypeStruct((n, c, hw), x_nchw.dtype),
        grid=(n // nb,),
        in_specs=[pl.BlockSpec((nb, c, hw), lambda b: (b, 0, 0))],
        out_specs=pl.BlockSpec((nb, c, hw), lambda b: (b, 0, 0)),
        compiler_params=pltpu.CompilerParams(
            dimension_semantics=("parallel",),
            vmem_limit_bytes=vmem_limit),
    )(x3)
    return y3.reshape(n, c, h, w)


def kernel(x_nchw):
    return _gct_forward(x_nchw)


# speedup vs baseline: 3.8945x; 3.7843x over previous
import functools

import jax
import jax.numpy as jnp
from jax.experimental import pallas as pl
from jax.experimental.pallas import tpu as pltpu


def _gct_block(x_ref, o_ref, *, c_gate, eps_s):
    x = x_ref[...].astype(jnp.float32)
    inv_c = 1.0 / x.shape[2]
    s = jnp.sum(x, axis=1, keepdims=True)
    m1 = jnp.sum(s, axis=2, keepdims=True) * inv_c
    m2 = jnp.sum(s * s, axis=2, keepdims=True) * inv_c
    var = jnp.maximum(m2 - m1 * m1, 0.0)
    z = (s - m1) * jax.lax.rsqrt(var + eps_s)
    o_ref[...] = (x * jnp.exp(z * z * (-0.5 * c_gate))).astype(o_ref.dtype)


@functools.partial(jax.jit, static_argnames=("c_gate", "eps"))
def _gct_forward(x_nchw, *, c_gate=2.0, eps=1e-5):
    n, c, h, w = x_nchw.shape
    hw = h * w
    x3 = jnp.transpose(x_nchw.reshape(n, c, hw), (0, 2, 1))
    slab = c * hw * jnp.dtype(x_nchw.dtype).itemsize

    budget = 40 << 20
    nb = 1
    for cand in range(n, 0, -1):
        if n % cand == 0 and 4 * cand * slab <= budget:
            nb = cand
            break

    vmem_limit = int(min(60 << 20, 4 * nb * slab + (8 << 20)))
    y3 = pl.pallas_call(
        functools.partial(_gct_block, c_gate=float(c_gate),
                          eps_s=float(eps) * float(hw) * float(hw)),
        out_shape=jax.ShapeDtypeStruct((n, hw, c), x_nchw.dtype),
        grid=(n // nb,),
        in_specs=[pl.BlockSpec((nb, hw, c), lambda b: (b, 0, 0))],
        out_specs=pl.BlockSpec((nb, hw, c), lambda b: (b, 0, 0)),
        compiler_params=pltpu.CompilerParams(
            dimension_semantics=("parallel",),
            vmem_limit_bytes=vmem_limit),
    )(x3)
    return jnp.transpose(y3, (0, 2, 1)).reshape(n, c, h, w)


def kernel(x_nchw):
    return _gct_forward(x_nchw)
